# baseline (speedup 1.0000x reference)
import jax
import jax.numpy as jnp
from jax.experimental import pallas as pl
from jax.experimental.pallas import tpu as pltpu

_N = 8192
_D = 768
_E = 8
_H = 128
_GH = 64
_OUT = 768
_TILE = 1024


def _moe_tile(x_ref, Wg1_ref, bg1_ref, Wg2_ref, bg2_ref,
              W1r_ref, b1r_ref, W2_ref, b2_ref, W3r_ref, b3_ref,
              out_ref, usage_ref, loss_ref):
    t = pl.program_id(0)
    x = x_ref[...]

    gh = jnp.maximum(
        jnp.dot(x, Wg1_ref[...], preferred_element_type=jnp.float32)
        + bg1_ref[...], 0.0)
    logits = jnp.dot(gh, Wg2_ref[...], preferred_element_type=jnp.float32) \
        + bg2_ref[...]
    m = jnp.max(logits, axis=-1, keepdims=True)
    ex = jnp.exp(logits - m)
    probs = ex / jnp.sum(ex, axis=-1, keepdims=True)

    idx = jax.lax.broadcasted_iota(jnp.int32, probs.shape, 1)
    p1 = jnp.max(probs, axis=-1, keepdims=True)
    i1 = jnp.min(jnp.where(probs >= p1, idx, _E), axis=-1, keepdims=True)
    oh1 = (idx == i1).astype(jnp.float32)
    probs2 = jnp.where(idx == i1, -jnp.inf, probs)
    p2 = jnp.max(probs2, axis=-1, keepdims=True)
    i2 = jnp.min(jnp.where(probs2 >= p2, idx, _E), axis=-1, keepdims=True)
    oh2 = (idx == i2).astype(jnp.float32)
    c = (oh1 * p1 + oh2 * p2) / (p1 + p2)

    cnt = jnp.sum(oh1 + oh2, axis=0, keepdims=True) * (1.0 / _N)

    @pl.when(t == 0)
    def _init():
        usage_ref[...] = cnt

    @pl.when(t > 0)
    def _acc():
        usage_ref[...] += cnt

    xb = x.astype(jnp.bfloat16)
    h1 = jnp.maximum(
        jnp.dot(xb, W1r_ref[...], preferred_element_type=jnp.float32)
        + b1r_ref[...], 0.0)
    parts = []
    for e in range(_E):
        h1e = h1[:, e * _H:(e + 1) * _H].astype(jnp.bfloat16)
        h2e = jnp.maximum(
            jnp.dot(h1e, W2_ref[e], preferred_element_type=jnp.float32)
            + b2_ref[e], 0.0)
        parts.append((h2e * c[:, e:e + 1]).astype(jnp.bfloat16))
    g = jnp.concatenate(parts, axis=1)
    out = jnp.dot(g, W3r_ref[...], preferred_element_type=jnp.float32)
    out = out + jnp.dot(c, b3_ref[...], preferred_element_type=jnp.float32)
    out_ref[...] = out

    @pl.when(t == pl.num_programs(0) - 1)
    def _loss():
        u = usage_ref[...]
        d = u - (1.0 / _E)
        loss_ref[...] = (jnp.sum(d * d) * (0.01 / _E)).reshape(1, 1)


def kernel(x, Wg1, bg1, Wg2, bg2, W1, b1, W2, b2, W3, b3):
    W1r = jnp.transpose(W1, (1, 0, 2)).reshape(_D, _E * _H).astype(jnp.bfloat16)
    b1r = b1.reshape(1, _E * _H)
    b2r = b2.reshape(_E, 1, _H)
    W3r = W3.reshape(_E * _H, _OUT).astype(jnp.bfloat16)
    W2b = W2.astype(jnp.bfloat16)

    grid = _N // _TILE
    out, usage, loss = pl.pallas_call(
        _moe_tile,
        grid=(grid,),
        in_specs=[
            pl.BlockSpec((_TILE, _D), lambda i: (i, 0)),
            pl.BlockSpec((_D, _GH), lambda i: (0, 0)),
            pl.BlockSpec((1, _GH), lambda i: (0, 0)),
            pl.BlockSpec((_GH, _E), lambda i: (0, 0)),
            pl.BlockSpec((1, _E), lambda i: (0, 0)),
            pl.BlockSpec((_D, _E * _H), lambda i: (0, 0)),
            pl.BlockSpec((1, _E * _H), lambda i: (0, 0)),
            pl.BlockSpec((_E, _H, _H), lambda i: (0, 0, 0)),
            pl.BlockSpec((_E, 1, _H), lambda i: (0, 0, 0)),
            pl.BlockSpec((_E * _H, _OUT), lambda i: (0, 0)),
            pl.BlockSpec((_E, _OUT), lambda i: (0, 0)),
        ],
        out_specs=[
            pl.BlockSpec((_TILE, _OUT), lambda i: (i, 0)),
            pl.BlockSpec((1, _E), lambda i: (0, 0)),
            pl.BlockSpec((1, 1), lambda i: (0, 0)),
        ],
        out_shape=[
            jax.ShapeDtypeStruct((_N, _OUT), jnp.float32),
            jax.ShapeDtypeStruct((1, _E), jnp.float32),
            jax.ShapeDtypeStruct((1, 1), jnp.float32),
        ],
        compiler_params=pltpu.CompilerParams(
            dimension_semantics=("arbitrary",),
        ),
    )(x, Wg1, bg1.reshape(1, _GH), Wg2, bg2.reshape(1, _E),
      W1r, b1r, W2b, b2r, W3r, b3)
    return out, loss[0, 0], usage.reshape(_E)


# speedup vs baseline: 1.1354x; 1.1354x over previous
import jax
import jax.numpy as jnp
from jax.experimental import pallas as pl
from jax.experimental.pallas import tpu as pltpu

_N = 8192
_D = 768
_E = 8
_H = 128
_GH = 64
_OUT = 768
_TILE = 1024
_GRID = _N // _TILE


def _moe_tile(x_ref, Wg1_ref, bg1_ref, Wg2_ref, bg2_ref,
              W1r_ref, b1r_ref, W2_ref, b2_ref, W3r_ref, b3_ref,
              out_ref, cnt_ref):
    x = x_ref[...]

    gh = jnp.maximum(
        jnp.dot(x, Wg1_ref[...], preferred_element_type=jnp.float32)
        + bg1_ref[...], 0.0)
    logits = jnp.dot(gh, Wg2_ref[...], preferred_element_type=jnp.float32) \
        + bg2_ref[...]
    m = jnp.max(logits, axis=-1, keepdims=True)
    ex = jnp.exp(logits - m)
    probs = ex / jnp.sum(ex, axis=-1, keepdims=True)

    idx = jax.lax.broadcasted_iota(jnp.int32, probs.shape, 1)
    p1 = jnp.max(probs, axis=-1, keepdims=True)
    i1 = jnp.min(jnp.where(probs >= p1, idx, _E), axis=-1, keepdims=True)
    oh1 = (idx == i1).astype(jnp.float32)
    probs2 = jnp.where(idx == i1, -jnp.inf, probs)
    p2 = jnp.max(probs2, axis=-1, keepdims=True)
    i2 = jnp.min(jnp.where(probs2 >= p2, idx, _E), axis=-1, keepdims=True)
    oh2 = (idx == i2).astype(jnp.float32)
    c = (oh1 * p1 + oh2 * p2) / (p1 + p2)

    cnt_ref[...] = (jnp.sum(oh1 + oh2, axis=0, keepdims=True)
                    * (1.0 / _N)).reshape(1, 1, _E)

    xb = x.astype(jnp.bfloat16)
    h1 = jnp.maximum(
        jnp.dot(xb, W1r_ref[...], preferred_element_type=jnp.float32)
        + b1r_ref[...], 0.0)
    parts = []
    for e in range(_E):
        h1e = h1[:, e * _H:(e + 1) * _H].astype(jnp.bfloat16)
        h2e = jnp.maximum(
            jnp.dot(h1e, W2_ref[e], preferred_element_type=jnp.float32)
            + b2_ref[e], 0.0)
        parts.append((h2e * c[:, e:e + 1]).astype(jnp.bfloat16))
    g = jnp.concatenate(parts, axis=1)
    out = jnp.dot(g, W3r_ref[...], preferred_element_type=jnp.float32)
    out = out + jnp.dot(c, b3_ref[...], preferred_element_type=jnp.float32)
    out_ref[...] = out


def _usage_loss(cnt_ref, usage_ref, loss_ref):
    u = jnp.sum(cnt_ref[...,  0, :], axis=0, keepdims=True)
    usage_ref[...] = u
    d = u - (1.0 / _E)
    loss_ref[...] = jnp.sum(d * d, axis=1, keepdims=True) * (0.01 / _E)


def kernel(x, Wg1, bg1, Wg2, bg2, W1, b1, W2, b2, W3, b3):
    W1r = jnp.transpose(W1, (1, 0, 2)).reshape(_D, _E * _H).astype(jnp.bfloat16)
    b1r = b1.reshape(1, _E * _H)
    b2r = b2.reshape(_E, 1, _H)
    W3r = W3.reshape(_E * _H, _OUT).astype(jnp.bfloat16)
    W2b = W2.astype(jnp.bfloat16)

    out, cnt = pl.pallas_call(
        _moe_tile,
        grid=(_GRID,),
        in_specs=[
            pl.BlockSpec((_TILE, _D), lambda i: (i, 0)),
            pl.BlockSpec((_D, _GH), lambda i: (0, 0)),
            pl.BlockSpec((1, _GH), lambda i: (0, 0)),
            pl.BlockSpec((_GH, _E), lambda i: (0, 0)),
            pl.BlockSpec((1, _E), lambda i: (0, 0)),
            pl.BlockSpec((_D, _E * _H), lambda i: (0, 0)),
            pl.BlockSpec((1, _E * _H), lambda i: (0, 0)),
            pl.BlockSpec((_E, _H, _H), lambda i: (0, 0, 0)),
            pl.BlockSpec((_E, 1, _H), lambda i: (0, 0, 0)),
            pl.BlockSpec((_E * _H, _OUT), lambda i: (0, 0)),
            pl.BlockSpec((_E, _OUT), lambda i: (0, 0)),
        ],
        out_specs=[
            pl.BlockSpec((_TILE, _OUT), lambda i: (i, 0)),
            pl.BlockSpec((1, 1, _E), lambda i: (i, 0, 0)),
        ],
        out_shape=[
            jax.ShapeDtypeStruct((_N, _OUT), jnp.float32),
            jax.ShapeDtypeStruct((_GRID, 1, _E), jnp.float32),
        ],
        compiler_params=pltpu.CompilerParams(
            dimension_semantics=("parallel",),
        ),
    )(x, Wg1, bg1.reshape(1, _GH), Wg2, bg2.reshape(1, _E),
      W1r, b1r, W2b, b2r, W3r, b3)

    usage, loss = pl.pallas_call(
        _usage_loss,
        out_shape=[
            jax.ShapeDtypeStruct((1, _E), jnp.float32),
            jax.ShapeDtypeStruct((1, 1), jnp.float32),
        ],
    )(cnt)
    return out, loss[0, 0], usage.reshape(_E)
